# baseline (speedup 1.0000x reference)
import functools

import numpy as np
import jax
import jax.numpy as jnp
from jax import lax
from jax.experimental import pallas as pl
from jax.experimental.pallas import tpu as pltpu
from jax.experimental.pallas import tpu_sc as plsc

_NC = 2
_NS = 16
_NW = _NC * _NS
_NB = 4


def _pe_table(seq_len, dmodel):
    position = np.arange(seq_len, dtype=np.float32)[:, None]
    div_term = np.exp(
        np.arange(0, dmodel, 2, dtype=np.float32)
        * (-np.log(np.float32(10000.0)) / np.float32(dmodel))
    )
    pe = np.zeros((seq_len, dmodel), dtype=np.float32)
    pe[:, 0::2] = np.sin(position * div_term)
    pe[:, 1::2] = np.cos(position * div_term)
    return jnp.asarray(np.stack([pe[0::2], pe[1::2]], axis=0))


@functools.lru_cache(maxsize=None)
def _make_sc_kernel(n_batch, seq_len, dmodel):
    assert n_batch % _NW == 0 and seq_len % 2 == 0
    half = seq_len // 2
    assert half <= 128
    seq_per_w = n_batch // _NW
    n_outer = seq_per_w // _NB
    assert seq_per_w % _NB == 0 and n_outer >= 2
    mesh = plsc.VectorSubcoreMesh(core_axis_name="c", subcore_axis_name="s")

    @functools.partial(
        pl.kernel,
        out_type=jax.ShapeDtypeStruct(
            (n_batch * half, 2 * dmodel), jnp.float32
        ),
        mesh=mesh,
        compiler_params=pltpu.CompilerParams(use_tc_tiling_on_sc=False),
        scratch_types=[
            pltpu.VMEM_SHARED((2, half, dmodel), jnp.float32),
            pltpu.VMEM((seq_per_w * 2, half), jnp.int32),
            pltpu.VMEM((_NB, 2, half, dmodel), jnp.float32),
            pltpu.SemaphoreType.DMA((_NB,)),
            pltpu.SemaphoreType.DMA((_NB,)),
            pltpu.SemaphoreType.DMA((_NB,)),
        ],
    )
    def run(x_hbm, pe_hbm, table_hbm, out_hbm, pe_v, idx_v, rows_v, psem,
            gsem, osem):
        sid = lax.axis_index("s")
        wid = sid * _NC + lax.axis_index("c")
        seq0 = wid * seq_per_w

        @pl.when(sid == 0)
        def _():
            pltpu.sync_copy(pe_hbm, pe_v)

        pltpu.sync_copy(
            x_hbm.at[pl.ds(seq0 * 2, seq_per_w * 2)], idx_v
        )
        plsc.subcore_barrier()

        def fire_pe(k):
            pltpu.async_copy(pe_v, rows_v.at[k], psem.at[k])

        def wait_pe(k):
            pltpu.make_async_copy(pe_v, rows_v.at[k], psem.at[k]).wait()

        def fire_gather(s, k):
            for par in range(2):
                pltpu.async_copy(
                    table_hbm.at[idx_v.at[s * 2 + par]],
                    rows_v.at[k].at[par],
                    gsem.at[k], add=True,
                )

        def wait_gather(k):
            for par in range(2):
                pltpu.make_async_copy(
                    table_hbm.at[pl.ds(0, half)],
                    rows_v.at[k].at[par], gsem.at[k],
                ).wait()

        def fire_out(s, k):
            for par in range(2):
                pltpu.async_copy(
                    rows_v.at[k].at[par],
                    out_hbm.at[pl.ds((seq0 + s) * half, half),
                               pl.ds(par * dmodel, dmodel)],
                    osem.at[k],
                )

        def wait_out(k):
            for par in range(2):
                pltpu.make_async_copy(
                    rows_v.at[k].at[par],
                    out_hbm.at[pl.ds(0, half), pl.ds(par * dmodel, dmodel)],
                    osem.at[k],
                ).wait()

        for kk in range(_NB):
            fire_pe(kk)
            if kk >= 1:
                wait_gather(kk - 1)
                fire_out(kk - 1, kk - 1)
            wait_pe(kk)
            fire_gather(kk, kk)

        def outer(g, carry):
            s_base = g * _NB
            for kk in range(_NB):
                s = s_base + kk
                wait_out(kk)
                fire_pe(kk)
                kp = (kk - 1) % _NB
                wait_gather(kp)
                fire_out(s - 1, kp)
                wait_pe(kk)
                fire_gather(s, kk)
            return carry

        lax.fori_loop(1, n_outer, outer, 0)

        wait_gather(_NB - 1)
        fire_out(seq_per_w - 1, _NB - 1)
        for kk in range(_NB):
            wait_out(kk)

    return run


def kernel(x, emb_table):
    n_batch, seq_len = x.shape
    vocab, dmodel = emb_table.shape
    pe = _pe_table(seq_len, dmodel)
    xi = x.astype(jnp.int32)
    x_de = jnp.stack([xi[:, 0::2], xi[:, 1::2]], axis=1)
    x_flat = x_de.reshape(n_batch * 2, seq_len // 2)
    tab2 = lax.optimization_barrier(
        emb_table.reshape(vocab // 2, 2 * dmodel)
    )
    tab_lin = tab2.reshape(vocab, dmodel)
    out = _make_sc_kernel(n_batch, seq_len, dmodel)(x_flat, pe, tab_lin)
    return out.reshape(n_batch, seq_len, dmodel)


# speedup vs baseline: 1.2296x; 1.2296x over previous
import functools

import numpy as np
import jax
import jax.numpy as jnp
from jax import lax
from jax.experimental import pallas as pl
from jax.experimental.pallas import tpu as pltpu
from jax.experimental.pallas import tpu_sc as plsc
from jax.experimental import layout as jex_layout

_NC = 2
_NS = 16
_NW = _NC * _NS
_NB = 4


def _pe_table(seq_len, dmodel):
    position = np.arange(seq_len, dtype=np.float32)[:, None]
    div_term = np.exp(
        np.arange(0, dmodel, 2, dtype=np.float32)
        * (-np.log(np.float32(10000.0)) / np.float32(dmodel))
    )
    pe = np.zeros((seq_len, dmodel), dtype=np.float32)
    pe[:, 0::2] = np.sin(position * div_term)
    pe[:, 1::2] = np.cos(position * div_term)
    return jnp.asarray(np.stack([pe[0::2], pe[1::2]], axis=0))


@functools.lru_cache(maxsize=None)
def _make_sc_kernel(n_batch, seq_len, dmodel):
    assert n_batch % _NW == 0 and seq_len % 2 == 0
    half = seq_len // 2
    assert half <= 128
    seq_per_w = n_batch // _NW
    n_outer = seq_per_w // _NB
    assert seq_per_w % _NB == 0 and n_outer >= 2
    mesh = plsc.VectorSubcoreMesh(core_axis_name="c", subcore_axis_name="s")

    @functools.partial(
        pl.kernel,
        out_type=jax.ShapeDtypeStruct(
            (n_batch * half, 2 * dmodel), jnp.float32
        ),
        mesh=mesh,
        compiler_params=pltpu.CompilerParams(use_tc_tiling_on_sc=False),
        scratch_types=[
            pltpu.VMEM_SHARED((2, half, dmodel), jnp.float32),
            pltpu.VMEM((seq_per_w * 2, half), jnp.int32),
            pltpu.VMEM((_NB, 2, half, dmodel), jnp.float32),
            pltpu.SemaphoreType.DMA((_NB,)),
            pltpu.SemaphoreType.DMA((_NB,)),
            pltpu.SemaphoreType.DMA((_NB,)),
        ],
    )
    def run(x_hbm, pe_hbm, table_hbm, out_hbm, pe_v, idx_v, rows_v, psem,
            gsem, osem):
        sid = lax.axis_index("s")
        wid = sid * _NC + lax.axis_index("c")
        seq0 = wid * seq_per_w

        @pl.when(sid == 0)
        def _():
            pltpu.sync_copy(pe_hbm, pe_v)

        pltpu.sync_copy(
            x_hbm.at[pl.ds(seq0 * 2, seq_per_w * 2)], idx_v
        )
        plsc.subcore_barrier()

        def fire_pe(k):
            pltpu.async_copy(pe_v, rows_v.at[k], psem.at[k])

        def wait_pe(k):
            pltpu.make_async_copy(pe_v, rows_v.at[k], psem.at[k]).wait()

        def fire_gather(s, k):
            for par in range(2):
                pltpu.async_copy(
                    table_hbm.at[idx_v.at[s * 2 + par]],
                    rows_v.at[k].at[par],
                    gsem.at[k], add=True,
                )

        def wait_gather(k):
            for par in range(2):
                pltpu.make_async_copy(
                    table_hbm.at[pl.ds(0, half)],
                    rows_v.at[k].at[par], gsem.at[k],
                ).wait()

        def fire_out(s, k):
            for par in range(2):
                pltpu.async_copy(
                    rows_v.at[k].at[par],
                    out_hbm.at[pl.ds((seq0 + s) * half, half),
                               pl.ds(par * dmodel, dmodel)],
                    osem.at[k],
                )

        def wait_out(k):
            for par in range(2):
                pltpu.make_async_copy(
                    rows_v.at[k].at[par],
                    out_hbm.at[pl.ds(0, half), pl.ds(par * dmodel, dmodel)],
                    osem.at[k],
                ).wait()

        for kk in range(_NB):
            fire_pe(kk)
            if kk >= 1:
                wait_gather(kk - 1)
                fire_out(kk - 1, kk - 1)
            wait_pe(kk)
            fire_gather(kk, kk)

        def outer(g, carry):
            s_base = g * _NB
            for kk in range(_NB):
                s = s_base + kk
                wait_out(kk)
                fire_pe(kk)
                kp = (kk - 1) % _NB
                wait_gather(kp)
                fire_out(s - 1, kp)
                wait_pe(kk)
                fire_gather(s, kk)
            return carry

        lax.fori_loop(1, n_outer, outer, 0)

        wait_gather(_NB - 1)
        fire_out(seq_per_w - 1, _NB - 1)
        for kk in range(_NB):
            wait_out(kk)

    return run


def kernel(x, emb_table):
    n_batch, seq_len = x.shape
    vocab, dmodel = emb_table.shape
    pe = _pe_table(seq_len, dmodel)
    xi = x.astype(jnp.int32) * 2
    x_de = jnp.stack([xi[:, 0::2], xi[:, 1::2]], axis=1)
    x_flat = x_de.reshape(n_batch * 2, seq_len // 2)
    emb_nat = jex_layout.with_layout_constraint(
        emb_table, jex_layout.Layout((0, 1), tiling=((8, 128),))
    )
    out = _make_sc_kernel(n_batch, seq_len, dmodel)(x_flat, pe, emb_nat)
    return out.reshape(n_batch, seq_len, dmodel)
